# baseline (speedup 1.0000x reference)
import dataclasses

import jax
import jax.numpy as jnp
from jax import lax
from jax.experimental import pallas as pl
from jax.experimental.pallas import tpu as pltpu
from jax.experimental.pallas import tpu_sc as plsc

_BATCH = 1024
_CH = 30
_NW = 32
_BW = 512
_SC_CH = 16
_UNITS = 14 * 14 * (_BATCH // _BW)
_L_COORD = 5.0
_L_NOOBJ = 0.5


def _sqrt_pos(x):
    i = plsc.bitcast(x, jnp.int32)
    i = jnp.int32(0x5F3759DF) - lax.shift_right_arithmetic(i, 1)
    y = plsc.bitcast(i, jnp.float32)
    y = y * (1.5 - 0.5 * x * y * y)
    y = y * (1.5 - 0.5 * x * y * y)
    y = y * (1.5 - 0.5 * x * y * y)
    return x * y


def _unit_coords(u):
    cell = u // 2
    h = u - cell * 2
    i = cell // 14
    j = cell - i * 14
    b0 = pl.multiple_of(h * _BW, 128)
    return i, j, b0


def _sc_body(xp_hbm, xt_hbm, o_hbm, pb0, tb0, pb1, tb1, out16, acc_ref,
             sem0, sem1, osem):
    wid = lax.axis_index("c") * 16 + lax.axis_index("s")
    nu = jnp.where(wid < 8, 13, 12)
    u_base = wid * 12 + jnp.minimum(wid, 8)
    acc_ref[...] = jnp.zeros((16,), jnp.float32)

    def issue(u, pbuf, tbuf, sem):
        i, j, b0 = _unit_coords(u)
        pltpu.make_async_copy(
            xp_hbm.at[i, j, pl.ds(0, _SC_CH), pl.ds(b0, _BW)], pbuf, sem).start()
        pltpu.make_async_copy(
            xt_hbm.at[i, j, pl.ds(0, _SC_CH), pl.ds(b0, _BW)], tbuf, sem).start()

    def wait(u, pbuf, tbuf, sem):
        i, j, b0 = _unit_coords(u)
        pltpu.make_async_copy(
            xp_hbm.at[i, j, pl.ds(0, _SC_CH), pl.ds(b0, _BW)], pbuf, sem).wait()
        pltpu.make_async_copy(
            xt_hbm.at[i, j, pl.ds(0, _SC_CH), pl.ds(b0, _BW)], tbuf, sem).wait()

    def compute(pbuf, tbuf):
        @pl.loop(0, _BW // 16)
        def _(g8):
            b0 = g8 * 16

            def cp(c):
                return pbuf[c, pl.ds(b0, 16)]

            def ct(c):
                return tbuf[c, pl.ds(b0, 16)]

            t4 = ct(4)
            coo = (t4 > 0.0).astype(jnp.float32)
            noo = 1.0 - coo

            p4 = cp(4)
            p9 = cp(9)
            t9 = ct(9)
            d4 = p4 - t4
            d9 = p9 - t9
            noo_term = noo * (d4 * d4 + d9 * d9)

            p0, p1, p2, p3 = cp(0), cp(1), cp(2), cp(3)
            p5, p6, p7, p8 = cp(5), cp(6), cp(7), cp(8)
            t0, t1, t2, t3 = ct(0), ct(1), ct(2), ct(3)

            t_min_x = t0 * 64.0 - t2 * 224.0
            t_max_x = t0 * 64.0 + t2 * 224.0
            t_min_y = t1 * 64.0 - t3 * 224.0
            t_max_y = t1 * 64.0 + t3 * 224.0
            t_area = (t_max_x - t_min_x) * (t_max_y - t_min_y)

            def iou(x, y, w, h):
                mnx = x * 64.0 - w * 224.0
                mxx = x * 64.0 + w * 224.0
                mny = y * 64.0 - h * 224.0
                mxy = y * 64.0 + h * 224.0
                iw = jnp.maximum(
                    jnp.minimum(mxx, t_max_x) - jnp.maximum(mnx, t_min_x), 0.0)
                ih = jnp.maximum(
                    jnp.minimum(mxy, t_max_y) - jnp.maximum(mny, t_min_y), 0.0)
                inter = iw * ih
                area = (mxx - mnx) * (mxy - mny)
                return inter / (area + t_area - inter)

            iou_a = iou(p0, p1, p2, p3)
            iou_b = iou(p5, p6, p7, p8)
            sel = iou_a >= iou_b
            max_iou = jnp.maximum(iou_a, iou_b)

            resp_c = jnp.where(sel, p4, p9)
            nrsp_c = jnp.where(sel, p9, p4)
            dcon = resp_c - max_iou
            contain = coo * dcon * dcon
            ncl = coo * nrsp_c * nrsp_c

            t5, t6 = ct(5), ct(6)
            rx = jnp.where(sel, p0, p5) - jnp.where(sel, t0, t5)
            ry = jnp.where(sel, p1, p6) - jnp.where(sel, t1, t6)
            loc = rx * rx + ry * ry

            t7, t8 = ct(7), ct(8)
            rw_p = jnp.where(sel, p2, p7)
            rw_t = jnp.where(sel, t2, t7)
            rh_p = jnp.where(sel, p3, p8)
            rh_t = jnp.where(sel, t3, t8)
            loc = loc + rw_p + rw_t - 2.0 * _sqrt_pos(rw_p * rw_t)
            loc = loc + rh_p + rh_t - 2.0 * _sqrt_pos(rh_p * rh_t)

            total = (_L_COORD * coo * loc + contain + ncl
                     + _L_NOOBJ * noo_term)
            acc_ref[...] = acc_ref[...] + total

    issue(u_base, pb0, tb0, sem0)

    @pl.loop(0, 12, step=2)
    def _(k):
        u = u_base + k
        issue(u + 1, pb1, tb1, sem1)
        wait(u, pb0, tb0, sem0)
        compute(pb0, tb0)

        @pl.when(k + 2 < nu)
        def _():
            issue(u + 2, pb0, tb0, sem0)

        wait(u + 1, pb1, tb1, sem1)
        compute(pb1, tb1)

    @pl.when(nu == 13)
    def _():
        wait(u_base + 12, pb0, tb0, sem0)
        compute(pb0, tb0)

    out16[...] = acc_ref[...] * (1.0 / _BATCH)
    pltpu.make_async_copy(out16, o_hbm.at[wid], osem).start()
    pltpu.make_async_copy(out16, o_hbm.at[wid], osem).wait()


def _tc_cls_body(p_ref, t_ref, o_ref):
    k = pl.program_id(0)
    p = p_ref[0, 0]
    t = t_ref[0, 0]
    coo = (t[4:5, :] > 0.0).astype(jnp.float32)
    d = p[10:30, :] - t[10:30, :]
    s = jnp.sum(d * d, axis=0, keepdims=True)
    partial = jnp.sum(s * coo) * (1.0 / _BATCH)
    prev = jnp.where(k == 0, 0.0, o_ref[0, 0])
    o_ref[0, 0] = prev + partial


def _tc_cls(xp, xt):
    out = pl.pallas_call(
        _tc_cls_body,
        grid=(14 * 14,),
        in_specs=[
            pl.BlockSpec((1, 1, _CH, _BATCH), lambda k: (k // 14, k % 14, 0, 0)),
            pl.BlockSpec((1, 1, _CH, _BATCH), lambda k: (k // 14, k % 14, 0, 0)),
        ],
        out_specs=pl.BlockSpec((1, 1), lambda k: (0, 0), memory_space=pltpu.SMEM),
        out_shape=jax.ShapeDtypeStruct((1, 1), jnp.float32),
    )(xp, xt)
    return out[0, 0]


def kernel(pred_tensor, target_tensor):
    xp = jnp.transpose(pred_tensor, (1, 2, 3, 0))
    xt = jnp.transpose(target_tensor, (1, 2, 3, 0))
    cp = pltpu.CompilerParams()
    if "needs_layout_passes" in pltpu.CompilerParams.__dataclass_fields__:
        cp = dataclasses.replace(cp, needs_layout_passes=False)
    cp = dataclasses.replace(cp, use_tc_tiling_on_sc=True)
    mesh = plsc.VectorSubcoreMesh(core_axis_name="c", subcore_axis_name="s")
    run = pl.kernel(
        _sc_body,
        out_type=jax.ShapeDtypeStruct((_NW, 16), jnp.float32),
        mesh=mesh,
        scratch_types=[
            pltpu.VMEM((_SC_CH, _BW), jnp.float32),
            pltpu.VMEM((_SC_CH, _BW), jnp.float32),
            pltpu.VMEM((_SC_CH, _BW), jnp.float32),
            pltpu.VMEM((_SC_CH, _BW), jnp.float32),
            pltpu.VMEM((16,), jnp.float32),
            pltpu.VMEM((16,), jnp.float32),
            pltpu.SemaphoreType.DMA,
            pltpu.SemaphoreType.DMA,
            pltpu.SemaphoreType.DMA,
        ],
        compiler_params=cp,
    )
    return jnp.sum(run(xp, xt)) + _tc_cls(xp, xt)


# speedup vs baseline: 1.0910x; 1.0910x over previous
import dataclasses

import jax
import jax.numpy as jnp
from jax import lax
from jax.experimental import pallas as pl
from jax.experimental.pallas import tpu as pltpu
from jax.experimental.pallas import tpu_sc as plsc

_BATCH = 1024
_CH = 30
_NW = 32
_BW = 512
_SC_CH = 16
_UNITS = 14 * 14 * (_BATCH // _BW)
_L_COORD = 5.0
_L_NOOBJ = 0.5


def _sqrt_pos(x):
    i = plsc.bitcast(x, jnp.int32)
    i = jnp.int32(0x5F3759DF) - lax.shift_right_arithmetic(i, 1)
    y = plsc.bitcast(i, jnp.float32)
    y = y * (1.5 - 0.5 * x * y * y)
    y = y * (1.5 - 0.5 * x * y * y)
    y = y * (1.5 - 0.5 * x * y * y)
    return x * y


def _unit_coords(u):
    cell = u // 2
    h = u - cell * 2
    i = cell // 14
    j = cell - i * 14
    b0 = pl.multiple_of(h * _BW, 128)
    return i, j, b0


def _sc_body(xp_hbm, xt_hbm, o_hbm, pb0, tb0, pb1, tb1, out16, acc_ref,
             sem0, sem1, osem):
    wid = lax.axis_index("c") * 16 + lax.axis_index("s")
    nu = jnp.where(wid < 8, 13, 12)
    u_base = wid * 12 + jnp.minimum(wid, 8)
    acc_ref[...] = jnp.zeros((16,), jnp.float32)

    def issue(u, pbuf, tbuf, sem):
        i, j, b0 = _unit_coords(u)
        pltpu.make_async_copy(
            xp_hbm.at[i, j, pl.ds(0, _SC_CH), pl.ds(b0, _BW)], pbuf, sem).start()
        pltpu.make_async_copy(
            xt_hbm.at[i, j, pl.ds(0, _SC_CH), pl.ds(b0, _BW)], tbuf, sem).start()

    def wait(u, pbuf, tbuf, sem):
        i, j, b0 = _unit_coords(u)
        pltpu.make_async_copy(
            xp_hbm.at[i, j, pl.ds(0, _SC_CH), pl.ds(b0, _BW)], pbuf, sem).wait()
        pltpu.make_async_copy(
            xt_hbm.at[i, j, pl.ds(0, _SC_CH), pl.ds(b0, _BW)], tbuf, sem).wait()

    def compute(pbuf, tbuf):
        @pl.loop(0, _BW // 16)
        def _(g8):
            b0 = g8 * 16

            def cp(c):
                return pbuf[c, pl.ds(b0, 16)]

            def ct(c):
                return tbuf[c, pl.ds(b0, 16)]

            t4 = ct(4)
            coo = (t4 > 0.0).astype(jnp.float32)
            noo = 1.0 - coo

            p4 = cp(4)
            p9 = cp(9)
            t9 = ct(9)
            d4 = p4 - t4
            d9 = p9 - t9
            noo_term = noo * (d4 * d4 + d9 * d9)

            p0, p1, p2, p3 = cp(0), cp(1), cp(2), cp(3)
            p5, p6, p7, p8 = cp(5), cp(6), cp(7), cp(8)
            t0, t1, t2, t3 = ct(0), ct(1), ct(2), ct(3)

            t_min_x = t0 * 64.0 - t2 * 224.0
            t_max_x = t0 * 64.0 + t2 * 224.0
            t_min_y = t1 * 64.0 - t3 * 224.0
            t_max_y = t1 * 64.0 + t3 * 224.0
            t_area = (t_max_x - t_min_x) * (t_max_y - t_min_y)

            def iou(x, y, w, h):
                mnx = x * 64.0 - w * 224.0
                mxx = x * 64.0 + w * 224.0
                mny = y * 64.0 - h * 224.0
                mxy = y * 64.0 + h * 224.0
                iw = jnp.maximum(
                    jnp.minimum(mxx, t_max_x) - jnp.maximum(mnx, t_min_x), 0.0)
                ih = jnp.maximum(
                    jnp.minimum(mxy, t_max_y) - jnp.maximum(mny, t_min_y), 0.0)
                inter = iw * ih
                area = (mxx - mnx) * (mxy - mny)
                return inter / (area + t_area - inter)

            iou_a = iou(p0, p1, p2, p3)
            iou_b = iou(p5, p6, p7, p8)
            sel = iou_a >= iou_b
            max_iou = jnp.maximum(iou_a, iou_b)

            resp_c = jnp.where(sel, p4, p9)
            nrsp_c = jnp.where(sel, p9, p4)
            dcon = resp_c - max_iou
            contain = coo * dcon * dcon
            ncl = coo * nrsp_c * nrsp_c

            t5, t6 = ct(5), ct(6)
            rx = jnp.where(sel, p0, p5) - jnp.where(sel, t0, t5)
            ry = jnp.where(sel, p1, p6) - jnp.where(sel, t1, t6)
            loc = rx * rx + ry * ry

            t7, t8 = ct(7), ct(8)
            rw_p = jnp.where(sel, p2, p7)
            rw_t = jnp.where(sel, t2, t7)
            rh_p = jnp.where(sel, p3, p8)
            rh_t = jnp.where(sel, t3, t8)
            loc = loc + rw_p + rw_t - 2.0 * _sqrt_pos(rw_p * rw_t)
            loc = loc + rh_p + rh_t - 2.0 * _sqrt_pos(rh_p * rh_t)

            total = (_L_COORD * coo * loc + contain + ncl
                     + _L_NOOBJ * noo_term)
            acc_ref[...] = acc_ref[...] + total

    issue(u_base, pb0, tb0, sem0)

    @pl.loop(0, 12, step=2)
    def _(k):
        u = u_base + k
        issue(u + 1, pb1, tb1, sem1)
        wait(u, pb0, tb0, sem0)
        compute(pb0, tb0)

        @pl.when(k + 2 < nu)
        def _():
            issue(u + 2, pb0, tb0, sem0)

        wait(u + 1, pb1, tb1, sem1)
        compute(pb1, tb1)

    @pl.when(nu == 13)
    def _():
        wait(u_base + 12, pb0, tb0, sem0)
        compute(pb0, tb0)

    out16[...] = acc_ref[...] * (1.0 / _BATCH)
    pltpu.make_async_copy(out16, o_hbm.at[wid], osem).start()
    pltpu.make_async_copy(out16, o_hbm.at[wid], osem).wait()


def _tc_cls_body(p_ref, t_ref, o_ref, acc_ref):
    k = pl.program_id(0)
    p = p_ref[0, 0]
    t = t_ref[0, 0]
    coo = (t[4:5, :] > 0.0).astype(jnp.float32)
    chm = (lax.broadcasted_iota(jnp.int32, (_CH, 1), 0) >= 10).astype(jnp.float32)
    d = p - t
    term = d * d * (coo * chm)

    @pl.when(k == 0)
    def _():
        acc_ref[...] = jnp.zeros((_CH, _BATCH), jnp.float32)

    acc_ref[...] = acc_ref[...] + term

    @pl.when(k == 14 * 14 - 1)
    def _():
        o_ref[0, 0] = jnp.sum(acc_ref[...]) * (1.0 / _BATCH)


def _tc_cls(xp, xt):
    out = pl.pallas_call(
        _tc_cls_body,
        grid=(14 * 14,),
        in_specs=[
            pl.BlockSpec((1, 1, _CH, _BATCH), lambda k: (k // 14, k % 14, 0, 0)),
            pl.BlockSpec((1, 1, _CH, _BATCH), lambda k: (k // 14, k % 14, 0, 0)),
        ],
        out_specs=pl.BlockSpec((1, 1), lambda k: (0, 0), memory_space=pltpu.SMEM),
        out_shape=jax.ShapeDtypeStruct((1, 1), jnp.float32),
        scratch_shapes=[pltpu.VMEM((_CH, _BATCH), jnp.float32)],
    )(xp, xt)
    return out[0, 0]


def kernel(pred_tensor, target_tensor):
    xp = jnp.transpose(pred_tensor, (1, 2, 3, 0))
    xt = jnp.transpose(target_tensor, (1, 2, 3, 0))
    cp = pltpu.CompilerParams()
    if "needs_layout_passes" in pltpu.CompilerParams.__dataclass_fields__:
        cp = dataclasses.replace(cp, needs_layout_passes=False)
    cp = dataclasses.replace(cp, use_tc_tiling_on_sc=True)
    mesh = plsc.VectorSubcoreMesh(core_axis_name="c", subcore_axis_name="s")
    run = pl.kernel(
        _sc_body,
        out_type=jax.ShapeDtypeStruct((_NW, 16), jnp.float32),
        mesh=mesh,
        scratch_types=[
            pltpu.VMEM((_SC_CH, _BW), jnp.float32),
            pltpu.VMEM((_SC_CH, _BW), jnp.float32),
            pltpu.VMEM((_SC_CH, _BW), jnp.float32),
            pltpu.VMEM((_SC_CH, _BW), jnp.float32),
            pltpu.VMEM((16,), jnp.float32),
            pltpu.VMEM((16,), jnp.float32),
            pltpu.SemaphoreType.DMA,
            pltpu.SemaphoreType.DMA,
            pltpu.SemaphoreType.DMA,
        ],
        compiler_params=cp,
    )
    return jnp.sum(run(xp, xt)) + _tc_cls(xp, xt)


# speedup vs baseline: 2.7398x; 2.5113x over previous
import dataclasses

import jax
import jax.numpy as jnp
from jax import lax
from jax.experimental import pallas as pl
from jax.experimental.pallas import tpu as pltpu
from jax.experimental.pallas import tpu_sc as plsc

_BATCH = 1024
_CH = 30
_NW = 32
_BW = 512
_UNITS = 14 * 14 * (_BATCH // _BW)
_L_COORD = 5.0
_L_NOOBJ = 0.5


def _sqrt_pos(x):
    i = plsc.bitcast(x, jnp.int32)
    i = jnp.int32(0x5F3759DF) - lax.shift_right_arithmetic(i, 1)
    y = plsc.bitcast(i, jnp.float32)
    y = y * (1.5 - 0.5 * x * y * y)
    y = y * (1.5 - 0.5 * x * y * y)
    y = y * (1.5 - 0.5 * x * y * y)
    return x * y


def _unit_coords(u):
    cell = u // 2
    h = u - cell * 2
    i = cell // 14
    j = cell - i * 14
    b0 = pl.multiple_of(h * _BW, 128)
    return i, j, b0


def _sc_body(xp_hbm, xt_hbm, o_hbm, pb0, tb0, pb1, tb1, out16, acc_ref,
             sem0, sem1, osem):
    wid = lax.axis_index("c") * 16 + lax.axis_index("s")
    nu = jnp.where(wid < 8, 13, 12)
    u_base = wid * 12 + jnp.minimum(wid, 8)
    acc_ref[...] = jnp.zeros((16,), jnp.float32)

    def issue(u, pbuf, tbuf, sem):
        i, j, b0 = _unit_coords(u)
        pltpu.make_async_copy(
            xp_hbm.at[i, j, :, pl.ds(b0, _BW)], pbuf, sem).start()
        pltpu.make_async_copy(
            xt_hbm.at[i, j, :, pl.ds(b0, _BW)], tbuf, sem).start()

    def wait(u, pbuf, tbuf, sem):
        i, j, b0 = _unit_coords(u)
        pltpu.make_async_copy(
            xp_hbm.at[i, j, :, pl.ds(b0, _BW)], pbuf, sem).wait()
        pltpu.make_async_copy(
            xt_hbm.at[i, j, :, pl.ds(b0, _BW)], tbuf, sem).wait()

    def compute(pbuf, tbuf):
        @pl.loop(0, _BW // 16)
        def _(g8):
            b0 = g8 * 16

            def cp(c):
                return pbuf[c, pl.ds(b0, 16)]

            def ct(c):
                return tbuf[c, pl.ds(b0, 16)]

            t4 = ct(4)
            coo = (t4 > 0.0).astype(jnp.float32)
            noo = 1.0 - coo

            p4 = cp(4)
            p9 = cp(9)
            t9 = ct(9)
            d4 = p4 - t4
            d9 = p9 - t9
            noo_term = noo * (d4 * d4 + d9 * d9)

            p0, p1, p2, p3 = cp(0), cp(1), cp(2), cp(3)
            p5, p6, p7, p8 = cp(5), cp(6), cp(7), cp(8)
            t0, t1, t2, t3 = ct(0), ct(1), ct(2), ct(3)

            t_min_x = t0 * 64.0 - t2 * 224.0
            t_max_x = t0 * 64.0 + t2 * 224.0
            t_min_y = t1 * 64.0 - t3 * 224.0
            t_max_y = t1 * 64.0 + t3 * 224.0
            t_area = (t_max_x - t_min_x) * (t_max_y - t_min_y)

            def iou(x, y, w, h):
                mnx = x * 64.0 - w * 224.0
                mxx = x * 64.0 + w * 224.0
                mny = y * 64.0 - h * 224.0
                mxy = y * 64.0 + h * 224.0
                iw = jnp.maximum(
                    jnp.minimum(mxx, t_max_x) - jnp.maximum(mnx, t_min_x), 0.0)
                ih = jnp.maximum(
                    jnp.minimum(mxy, t_max_y) - jnp.maximum(mny, t_min_y), 0.0)
                inter = iw * ih
                area = (mxx - mnx) * (mxy - mny)
                return inter / (area + t_area - inter)

            iou_a = iou(p0, p1, p2, p3)
            iou_b = iou(p5, p6, p7, p8)
            sel = iou_a >= iou_b
            max_iou = jnp.maximum(iou_a, iou_b)

            resp_c = jnp.where(sel, p4, p9)
            nrsp_c = jnp.where(sel, p9, p4)
            dcon = resp_c - max_iou
            contain = coo * dcon * dcon
            ncl = coo * nrsp_c * nrsp_c

            t5, t6 = ct(5), ct(6)
            rx = jnp.where(sel, p0, p5) - jnp.where(sel, t0, t5)
            ry = jnp.where(sel, p1, p6) - jnp.where(sel, t1, t6)
            loc = rx * rx + ry * ry

            t7, t8 = ct(7), ct(8)
            rw_p = jnp.where(sel, p2, p7)
            rw_t = jnp.where(sel, t2, t7)
            rh_p = jnp.where(sel, p3, p8)
            rh_t = jnp.where(sel, t3, t8)
            loc = loc + rw_p + rw_t - 2.0 * _sqrt_pos(rw_p * rw_t)
            loc = loc + rh_p + rh_t - 2.0 * _sqrt_pos(rh_p * rh_t)

            cls = jnp.zeros((16,), jnp.float32)
            for c in range(10, 30):
                d = cp(c) - ct(c)
                cls = cls + d * d

            total = (_L_COORD * coo * loc + contain + ncl
                     + _L_NOOBJ * noo_term + coo * cls)
            acc_ref[...] = acc_ref[...] + total

    issue(u_base, pb0, tb0, sem0)

    @pl.loop(0, 12, step=2)
    def _(k):
        u = u_base + k
        issue(u + 1, pb1, tb1, sem1)
        wait(u, pb0, tb0, sem0)
        compute(pb0, tb0)

        @pl.when(k + 2 < nu)
        def _():
            issue(u + 2, pb0, tb0, sem0)

        wait(u + 1, pb1, tb1, sem1)
        compute(pb1, tb1)

    @pl.when(nu == 13)
    def _():
        wait(u_base + 12, pb0, tb0, sem0)
        compute(pb0, tb0)

    out16[...] = acc_ref[...] * (1.0 / _BATCH)
    pltpu.make_async_copy(out16, o_hbm.at[wid], osem).start()
    pltpu.make_async_copy(out16, o_hbm.at[wid], osem).wait()


def kernel(pred_tensor, target_tensor):
    xp = jnp.transpose(pred_tensor, (1, 2, 3, 0))
    xt = jnp.transpose(target_tensor, (1, 2, 3, 0))
    cp = pltpu.CompilerParams()
    if "needs_layout_passes" in pltpu.CompilerParams.__dataclass_fields__:
        cp = dataclasses.replace(cp, needs_layout_passes=False)
    cp = dataclasses.replace(cp, use_tc_tiling_on_sc=True)
    mesh = plsc.VectorSubcoreMesh(core_axis_name="c", subcore_axis_name="s")
    run = pl.kernel(
        _sc_body,
        out_type=jax.ShapeDtypeStruct((_NW, 16), jnp.float32),
        mesh=mesh,
        scratch_types=[
            pltpu.VMEM((_CH, _BW), jnp.float32),
            pltpu.VMEM((_CH, _BW), jnp.float32),
            pltpu.VMEM((_CH, _BW), jnp.float32),
            pltpu.VMEM((_CH, _BW), jnp.float32),
            pltpu.VMEM((16,), jnp.float32),
            pltpu.VMEM((16,), jnp.float32),
            pltpu.SemaphoreType.DMA,
            pltpu.SemaphoreType.DMA,
            pltpu.SemaphoreType.DMA,
        ],
        compiler_params=cp,
    )
    return jnp.sum(run(xp, xt))
